# baseline (speedup 1.0000x reference)
import functools

import jax
import jax.numpy as jnp
from jax import lax
from jax.experimental import pallas as pl
from jax.experimental.pallas import tpu as pltpu
from jax.experimental.pallas import tpu_sc as plsc

H = 128
HH = H // 2
NC = 2
NS = 16
NW = NC * NS
LANES = 16

B, L = 4096, 200
BL = B * L
PER_W = BL // NW
P = 128
CHUNKS = PER_W // P
NCOMB = 120
UNROLL = 2


def _sc_kernel_body(x_hbm, ov_hbm, bl_hbm, comb_hbm, out_hbm,
                    xbuf0, xbuf1, ovidx0, ovidx1,
                    blidx0, blidx1, cidx0, cidx1, comb_v,
                    cidx_s0, cidx_s1,
                    sx0, sx1, si0, si1, so0, so1):
    wid = lax.axis_index("s") * NC + lax.axis_index("c")
    w0 = wid * PER_W

    xbuf = (xbuf0, xbuf1)
    ovidx = (ovidx0, ovidx1)
    blidx = (blidx0, blidx1)
    cidx = (cidx0, cidx1)
    cidx_s = (cidx_s0, cidx_s1)
    sx = (sx0, sx1)
    si = (si0, si1)
    so = (so0, so1)

    pltpu.sync_copy(comb_hbm, comb_v)

    iota = lax.iota(jnp.int32, LANES)

    def fire_idx(g, b):
        base = w0 + g * P
        pltpu.async_copy(ov_hbm.at[pl.ds(base, P)], ovidx[b], si[b])
        pltpu.async_copy(bl_hbm.at[pl.ds(base, P)], blidx[b], si[b])

    def wait_idx(g, b):
        base = w0 + g * P
        pltpu.make_async_copy(ov_hbm.at[pl.ds(base, P)], ovidx[b], si[b]).wait()
        pltpu.make_async_copy(bl_hbm.at[pl.ds(base, P)], blidx[b], si[b]).wait()

    def fire_x(g, b):
        base = w0 + g * P
        pltpu.async_copy(x_hbm.at[pl.ds(base * H, P * H)], xbuf[b], sx[b])

    def wait_x(g, b):
        base = w0 + g * P
        pltpu.make_async_copy(
            x_hbm.at[pl.ds(base * H, P * H)], xbuf[b], sx[b]).wait()

    def fire_out(g, b):
        base = w0 + g * P
        pltpu.async_copy(xbuf[b], out_hbm.at[pl.ds(base * H, P * H)], so[b])

    def wait_out(g, b):
        base = w0 + g * P
        pltpu.make_async_copy(
            xbuf[b], out_hbm.at[pl.ds(base * H, P * H)], so[b]).wait()

    fire_idx(0, 0)
    fire_idx(1, 1)
    fire_x(0, 0)

    def pair_body(h, carry):
        for b in range(2):
            g = h * 2 + b
            b1 = 1 - b

            wait_idx(g, b)

            def idx_body(v, c2):
                o = v * LANES
                cidx[b][pl.ds(o, LANES)] = (
                    ovidx[b][pl.ds(o, LANES)] * (6 * H)
                    + blidx[b][pl.ds(o, LANES)] * H
                )
                return c2

            lax.fori_loop(0, P // LANES, idx_body, 0)

            @pl.when(g + 2 < CHUNKS)
            def _pf_idx():
                fire_idx(g + 2, b)

            @pl.when(g + 1 < CHUNKS)
            def _pf_x():
                @pl.when(g >= 1)
                def _drain_prev_out():
                    wait_out(g - 1, b1)

                fire_x(g + 1, b1)

            wait_x(g, b)

            def pos_body(i, c2):
                cvec = cidx[b][pl.ds(i * LANES, LANES)]
                for u in range(LANES):
                    c = cvec[u]
                    pH = i * LANES * H + u * H
                    for j in range(H // LANES):
                        o = pH + j * LANES
                        val = comb_v[pl.ds(c + (j * LANES), LANES)]
                        plsc.addupdate(xbuf[b].at[pl.ds(o, LANES)], val)
                return c2

            lax.fori_loop(0, P // LANES, pos_body, 0)
            fire_out(g, b)
        return carry

    lax.fori_loop(0, CHUNKS // 2, pair_body, 0)

    wait_out(CHUNKS - 2, 0)
    wait_out(CHUNKS - 1, 1)


@jax.jit
def _run(x_flat, ov_flat, bl_flat, comb_table):
    mesh = plsc.VectorSubcoreMesh(core_axis_name="c", subcore_axis_name="s")
    k = functools.partial(
        pl.kernel,
        mesh=mesh,
        out_type=jax.ShapeDtypeStruct((BL * H,), jnp.float32),
        scratch_types=[
            pltpu.VMEM((P * H,), jnp.float32),
            pltpu.VMEM((P * H,), jnp.float32),
            pltpu.VMEM((P,), jnp.int32),
            pltpu.VMEM((P,), jnp.int32),
            pltpu.VMEM((P,), jnp.int32),
            pltpu.VMEM((P,), jnp.int32),
            pltpu.VMEM((P,), jnp.int32),
            pltpu.VMEM((P,), jnp.int32),
            pltpu.VMEM((NCOMB * H,), jnp.float32),
            pltpu.SMEM((P,), jnp.int32),
            pltpu.SMEM((P,), jnp.int32),
            pltpu.SemaphoreType.DMA,
            pltpu.SemaphoreType.DMA,
            pltpu.SemaphoreType.DMA,
            pltpu.SemaphoreType.DMA,
            pltpu.SemaphoreType.DMA,
            pltpu.SemaphoreType.DMA,
        ],
    )(_sc_kernel_body)
    return k(x_flat, ov_flat, bl_flat, comb_table)


def kernel(x, overs, balls_in_over, over_table, ball_table):
    x_flat = x.reshape(BL * H)
    ov_flat = overs.reshape(BL).astype(jnp.int32)
    bl_flat = balls_in_over.reshape(BL).astype(jnp.int32)
    comb = jnp.concatenate(
        [jnp.repeat(over_table, 6, axis=0),
         jnp.tile(ball_table, (over_table.shape[0], 1))],
        axis=-1,
    ).reshape(NCOMB * H)
    out = _run(x_flat, ov_flat, bl_flat, comb)
    return out.reshape(B, L, H)


# speedup vs baseline: 2.1263x; 2.1263x over previous
import functools

import jax
import jax.numpy as jnp
from jax import lax
from jax.experimental import pallas as pl
from jax.experimental.pallas import tpu as pltpu
from jax.experimental.pallas import tpu_sc as plsc

H = 128
HH = H // 2
NC = 2
NS = 16
NW = NC * NS
LANES = 16

B, L = 4096, 200
BL = B * L
PER_W = BL // NW
P = 128
CHUNKS = PER_W // P
NCOMB = 120
UNROLL = 2


def _sc_kernel_body(x_hbm, ov_hbm, bl_hbm, comb_hbm, out_hbm,
                    xbuf0, xbuf1, rows0, rows1, ovidx0, ovidx1,
                    blidx0, blidx1, cidx0, cidx1, comb_v,
                    sx0, sx1, si0, si1, so0, so1, sg):
    wid = lax.axis_index("s") * NC + lax.axis_index("c")
    w0 = wid * PER_W

    xbuf = (xbuf0, xbuf1)
    rows = (rows0, rows1)
    ovidx = (ovidx0, ovidx1)
    blidx = (blidx0, blidx1)
    cidx = (cidx0, cidx1)
    sx = (sx0, sx1)
    si = (si0, si1)
    so = (so0, so1)

    @pl.when(lax.axis_index("s") == 0)
    def _copy_table():
        pltpu.sync_copy(comb_hbm, comb_v)

    plsc.subcore_barrier()

    def fire_idx(g, b):
        base = w0 + g * P
        pltpu.async_copy(ov_hbm.at[pl.ds(base, P)], ovidx[b], si[b])
        pltpu.async_copy(bl_hbm.at[pl.ds(base, P)], blidx[b], si[b])

    def wait_idx(g, b):
        base = w0 + g * P
        pltpu.make_async_copy(ov_hbm.at[pl.ds(base, P)], ovidx[b], si[b]).wait()
        pltpu.make_async_copy(bl_hbm.at[pl.ds(base, P)], blidx[b], si[b]).wait()

    def fire_x(g, b):
        base = w0 + g * P
        pltpu.async_copy(x_hbm.at[pl.ds(base * H, P * H)], xbuf[b], sx[b])

    def wait_x(g, b):
        base = w0 + g * P
        pltpu.make_async_copy(
            x_hbm.at[pl.ds(base * H, P * H)], xbuf[b], sx[b]).wait()

    def fire_out(g, b):
        base = w0 + g * P
        pltpu.async_copy(xbuf[b], out_hbm.at[pl.ds(base * H, P * H)], so[b])

    def wait_out(g, b):
        base = w0 + g * P
        pltpu.make_async_copy(
            xbuf[b], out_hbm.at[pl.ds(base * H, P * H)], so[b]).wait()

    fire_idx(0, 0)
    fire_idx(1, 1)
    fire_x(0, 0)

    def pair_body(h, carry):
        for b in range(2):
            g = h * 2 + b
            b1 = 1 - b

            wait_idx(g, b)

            def idx_body(v, c2):
                o = v * LANES
                cidx[b][pl.ds(o, LANES)] = (
                    ovidx[b][pl.ds(o, LANES)] * 6 + blidx[b][pl.ds(o, LANES)]
                )
                return c2

            lax.fori_loop(0, P // LANES, idx_body, 0)
            gather = pltpu.async_copy(comb_v.at[cidx[b]], rows[b], sg)

            @pl.when(g + 2 < CHUNKS)
            def _pf_idx():
                fire_idx(g + 2, b)

            @pl.when(g + 1 < CHUNKS)
            def _pf_x():
                @pl.when(g >= 1)
                def _drain_prev_out():
                    wait_out(g - 1, b1)

                fire_x(g + 1, b1)

            wait_x(g, b)
            gather.wait()

            def pos_body(i, c2):
                for u in range(UNROLL):
                    p = i * UNROLL + u
                    for j in range(H // LANES):
                        o = p * H + j * LANES
                        plsc.addupdate(
                            xbuf[b].at[pl.ds(o, LANES)],
                            rows[b][p, pl.ds(j * LANES, LANES)],
                        )
                return c2

            lax.fori_loop(0, P // UNROLL, pos_body, 0)
            fire_out(g, b)
        return carry

    lax.fori_loop(0, CHUNKS // 2, pair_body, 0)

    wait_out(CHUNKS - 2, 0)
    wait_out(CHUNKS - 1, 1)


@jax.jit
def _run(x_flat, ov_flat, bl_flat, comb_table):
    mesh = plsc.VectorSubcoreMesh(core_axis_name="c", subcore_axis_name="s")
    k = functools.partial(
        pl.kernel,
        mesh=mesh,
        out_type=jax.ShapeDtypeStruct((BL * H,), jnp.float32),
        scratch_types=[
            pltpu.VMEM((P * H,), jnp.float32),
            pltpu.VMEM((P * H,), jnp.float32),
            pltpu.VMEM((P, H), jnp.float32),
            pltpu.VMEM((P, H), jnp.float32),
            pltpu.VMEM((P,), jnp.int32),
            pltpu.VMEM((P,), jnp.int32),
            pltpu.VMEM((P,), jnp.int32),
            pltpu.VMEM((P,), jnp.int32),
            pltpu.VMEM((P,), jnp.int32),
            pltpu.VMEM((P,), jnp.int32),
            pltpu.VMEM_SHARED((NCOMB, H), jnp.float32),
            pltpu.SemaphoreType.DMA,
            pltpu.SemaphoreType.DMA,
            pltpu.SemaphoreType.DMA,
            pltpu.SemaphoreType.DMA,
            pltpu.SemaphoreType.DMA,
            pltpu.SemaphoreType.DMA,
            pltpu.SemaphoreType.DMA,
        ],
    )(_sc_kernel_body)
    return k(x_flat, ov_flat, bl_flat, comb_table)


def kernel(x, overs, balls_in_over, over_table, ball_table):
    x_flat = x.reshape(BL * H)
    ov_flat = overs.reshape(BL).astype(jnp.int32)
    bl_flat = balls_in_over.reshape(BL).astype(jnp.int32)
    comb = jnp.concatenate(
        [jnp.repeat(over_table, 6, axis=0),
         jnp.tile(ball_table, (over_table.shape[0], 1))],
        axis=-1,
    )
    out = _run(x_flat, ov_flat, bl_flat, comb)
    return out.reshape(B, L, H)
